# baseline (speedup 1.0000x reference)
import functools

import jax
import jax.numpy as jnp
from jax import lax
from jax.experimental import pallas as pl
from jax.experimental.pallas import tpu as pltpu
from jax.experimental.pallas import tpu_sc as plsc

BN = 2048


def _exact_row_reduce(x):
    bn, D = x.shape
    rg = bn // 128
    rows = []
    for r in range(rg):
        s = None
        for g in range(D // 128):
            xt = x[128 * r:128 * r + 128, 128 * g:128 * g + 128].T
            v = xt.reshape(16, 8, 128)
            acc = v[0]
            for o in range(1, 16):
                acc = acc + v[o]
            a = acc[0:4] + acc[4:8]
            a = a[0:2] + a[2:4]
            a = a[0:1] + a[1:2]
            s = a if s is None else s + a
        rows.append(s)
    return jnp.concatenate(rows, axis=0)


def _phase1_body(c_ref, q_ref, m_ref, t_ref, d_ref, r_ref):
    c = c_ref[0]
    q = q_ref[0]
    m = m_ref[0]
    t = t_ref[0]
    diff = c - q
    p = diff * (diff * m)
    dsq = _exact_row_reduce(p)
    x = jnp.maximum(dsq, 1e-8)
    d = x * lax.rsqrt(x)
    d_ref[0, 0] = d
    u = diff * t
    u4 = (u[:, 0:128] + u[:, 128:256]) + (u[:, 256:384] + u[:, 384:512])
    lane = lax.broadcasted_iota(jnp.int32, (128, 8), 0)
    col = lax.broadcasted_iota(jnp.int32, (128, 8), 1)
    sel = (col == 0).astype(jnp.float32) * jnp.where(lane >= 0, 1.0, 0.0)
    tcv = lax.dot_general(u4, sel, (((1,), (0,)), ((), ())),
                          preferred_element_type=jnp.float32)
    r_ref[0, 0] = tcv[:, 0:1]


def _phase1(cand, q3, m3, t3):
    B, N, D = cand.shape
    nb = N // BN
    rg = BN // 128
    return pl.pallas_call(
        _phase1_body,
        grid=(B, nb),
        in_specs=[
            pl.BlockSpec((1, BN, D), lambda b, j: (b, j, 0)),
            pl.BlockSpec((1, 1, D), lambda b, j: (b, 0, 0)),
            pl.BlockSpec((1, 1, D), lambda b, j: (b, 0, 0)),
            pl.BlockSpec((1, 1, D), lambda b, j: (b, 0, 0)),
        ],
        out_specs=[
            pl.BlockSpec((1, 1, rg, 128), lambda b, j: (b, j, 0, 0)),
            pl.BlockSpec((1, 1, BN, 1), lambda b, j: (b, j, 0, 0)),
        ],
        out_shape=[
            jax.ShapeDtypeStruct((B, nb, rg, 128), jnp.float32),
            jax.ShapeDtypeStruct((B, nb, BN, 1), jnp.float32),
        ],
    )(cand, q3, m3, t3)


def _phase2_body(n_total, d_ref, r_ref, m_ref, t_ref, w_ref, b_ref,
                 scores_ref, idx_ref, gidx_ref, x_ref):
    nb, bn = x_ref.shape
    dist2d = d_ref[0].reshape(nb, bn)
    corr2d = dist2d + 0.1 * r_ref[0].reshape(nb, bn)
    x_ref[...] = dist2d

    row_i = lax.broadcasted_iota(jnp.int32, (nb, bn), 0)
    col_i = lax.broadcasted_iota(jnp.int32, (nb, bn), 1)
    gidx2d = row_i * bn + col_i
    lane32 = lax.broadcasted_iota(jnp.int32, (1, 32), 1)
    INF = jnp.float32(jnp.inf)
    BIGI = jnp.int32(2**30)

    def body(k, carry):
        vacc, iacc, cacc = carry
        x = x_ref[...]
        mv = jnp.min(x)
        ci = jnp.min(jnp.where(x == mv, gidx2d, BIGI))
        sel = gidx2d == ci
        cv = jnp.sum(jnp.where(sel, corr2d, 0.0))
        onehot = lane32 == k
        vacc = jnp.where(onehot, mv, vacc)
        iacc = jnp.where(onehot, ci, iacc)
        cacc = jnp.where(onehot, cv, cacc)
        x_ref[...] = jnp.where(sel, INF, x)
        return vacc, iacc, cacc

    vacc = jnp.zeros((1, 32), jnp.float32)
    iacc = jnp.zeros((1, 32), jnp.int32)
    cacc = jnp.zeros((1, 32), jnp.float32)
    vacc, iacc, cacc = lax.fori_loop(0, 32, body, (vacc, iacc, cacc))

    m_row = m_ref[0]
    t_row = t_ref[0]
    D = m_row.shape[1]
    w = w_ref[...]
    bs = (jnp.sum(w[:, :D] * m_row, axis=1)
          + jnp.sum(w[:, D:] * t_row, axis=1) + b_ref[0])
    iota6 = lax.iota(jnp.int32, 6)

    def bs_at(j):
        return jnp.sum(jnp.where(iota6 == j, bs, 0.0))

    d16 = dist2d[0:1, 0:16]
    c16 = corr2d[0:1, 0:16]

    eps = jnp.float32(0.001)
    ch0_near = 1.0 / (vacc + eps)
    ch1_near = 1.0 / (jnp.abs(cacc) + eps)
    ch0_16 = 1.0 / (d16 + eps)
    ch1_16 = 1.0 / (jnp.abs(c16) + eps)

    ch0 = jnp.concatenate([ch0_near, ch0_16, ch0_16], axis=1)
    ch1 = jnp.concatenate([ch1_near, ch1_16, ch1_16], axis=1)
    ones64 = jnp.ones((1, 64), jnp.float32)
    ch2 = ones64 * bs_at(2)
    ch3 = ones64 * bs_at(3)
    ch4 = ones64 * bs_at(4)
    ch5 = lax.broadcasted_iota(jnp.int32, (1, 64), 1).astype(jnp.float32) * (
        1.0 / 64.0)

    scores = jnp.concatenate(
        [ch[0, :, None] for ch in (ch0, ch1, ch2, ch3, ch4, ch5)], axis=1)
    scores_ref[0] = scores

    iota16 = lax.broadcasted_iota(jnp.int32, (1, 16), 1)
    idx64 = jnp.concatenate([iacc, iota16, iota16], axis=1)
    idx_ref[0, 0] = idx64[0]
    gidx_ref[0, 0] = idx64[0] + pl.program_id(0) * n_total


def _phase2(dist4, corr4, m3, t3, W, b2):
    B, nb0, rg, bn = dist4.shape
    nb = nb0 * rg
    D = m3.shape[2]
    N = nb * bn
    return pl.pallas_call(
        functools.partial(_phase2_body, N),
        grid=(B,),
        in_specs=[
            pl.BlockSpec((1, nb0, rg, bn), lambda b: (b, 0, 0, 0)),
            pl.BlockSpec((1, nb0, rg, bn), lambda b: (b, 0, 0, 0)),
            pl.BlockSpec((1, 1, D), lambda b: (b, 0, 0)),
            pl.BlockSpec((1, 1, D), lambda b: (b, 0, 0)),
            pl.BlockSpec((6, 2 * D), lambda b: (0, 0)),
            pl.BlockSpec((1, 6), lambda b: (0, 0)),
        ],
        out_specs=[
            pl.BlockSpec((1, 64, 6), lambda b: (b, 0, 0)),
            pl.BlockSpec((1, 1, 64), lambda b: (b, 0, 0)),
            pl.BlockSpec((1, 1, 64), lambda b: (b, 0, 0)),
        ],
        out_shape=[
            jax.ShapeDtypeStruct((B, 64, 6), jnp.float32),
            jax.ShapeDtypeStruct((B, 1, 64), jnp.int32),
            jax.ShapeDtypeStruct((B, 1, 64), jnp.int32),
        ],
        scratch_shapes=[pltpu.VMEM((nb, bn), jnp.float32)],
    )(dist4, corr4, m3, t3, W, b2)


def _sc_gather(table, gidx):
    G = gidx.shape[0]
    D = table.shape[1]
    info = plsc.get_sparse_core_info()
    nc, ns = info.num_cores, info.num_subcores
    nw = nc * ns
    bpw = G // nw
    mesh = plsc.VectorSubcoreMesh(core_axis_name="c", subcore_axis_name="s")

    @functools.partial(
        pl.kernel, mesh=mesh,
        out_type=jax.ShapeDtypeStruct((G, D), jnp.float32),
        scratch_types=[
            pltpu.VMEM((bpw,), jnp.int32),
            pltpu.VMEM((bpw, D), jnp.float32),
            pltpu.SemaphoreType.DMA,
        ],
    )
    def k(table_hbm, idx_hbm, out_hbm, idx_v, rows_v, sem):
        wid = lax.axis_index("s") * nc + lax.axis_index("c")
        base = wid * bpw
        pltpu.sync_copy(idx_hbm.at[pl.ds(base, bpw)], idx_v)
        pltpu.async_copy(table_hbm.at[idx_v], rows_v, sem).wait()
        pltpu.sync_copy(rows_v, out_hbm.at[pl.ds(base, bpw)])

    return k(table, gidx)


def kernel(query_embedding, candidate_embeddings, metric, transport, W, b):
    B, N, D = candidate_embeddings.shape
    q3 = query_embedding.reshape(B, 1, D)
    m3 = metric.reshape(B, 1, D)
    t3 = transport.reshape(B, 1, D)
    b2 = b.reshape(1, 6)

    dist4, tc4 = _phase1(candidate_embeddings, q3, m3, t3)
    tc4 = tc4.reshape(dist4.shape)
    scores, idx3, gidx3 = _phase2(dist4, tc4, m3, t3, W, b2)

    flat = candidate_embeddings.reshape(B * N, D)
    rows = _sc_gather(flat, gidx3.reshape(B * 64))
    selected_embeddings = rows.reshape(B, 64, D)
    selected_indices = idx3.reshape(B, 64)
    return selected_embeddings, scores, selected_indices


# speedup vs baseline: 1.0991x; 1.0991x over previous
import functools

import jax
import jax.numpy as jnp
from jax import lax
from jax.experimental import pallas as pl
from jax.experimental.pallas import tpu as pltpu
from jax.experimental.pallas import tpu_sc as plsc

BN = 4096


def _exact_row_reduce(x):
    bn, D = x.shape
    rg = bn // 128
    rows = []
    for r in range(rg):
        s = None
        for g in range(D // 128):
            xt = x[128 * r:128 * r + 128, 128 * g:128 * g + 128].T
            v = xt.reshape(16, 8, 128)
            acc = v[0]
            for o in range(1, 16):
                acc = acc + v[o]
            a = acc[0:4] + acc[4:8]
            a = a[0:2] + a[2:4]
            a = a[0:1] + a[1:2]
            s = a if s is None else s + a
        rows.append(s)
    return jnp.concatenate(rows, axis=0)


def _phase1_body(c_ref, q_ref, m_ref, t_ref, d_ref, r_ref):
    c = c_ref[0]
    q = q_ref[0]
    m = m_ref[0]
    t = t_ref[0]
    diff = c - q
    p = diff * (diff * m)
    dsq = _exact_row_reduce(p)
    x = jnp.maximum(dsq, 1e-8)
    d = x * lax.rsqrt(x)
    d_ref[0, 0] = d
    u = diff * t
    u4 = (u[:, 0:128] + u[:, 128:256]) + (u[:, 256:384] + u[:, 384:512])
    lane = lax.broadcasted_iota(jnp.int32, (128, 8), 0)
    col = lax.broadcasted_iota(jnp.int32, (128, 8), 1)
    sel = (col == 0).astype(jnp.float32) * jnp.where(lane >= 0, 1.0, 0.0)
    tcv = lax.dot_general(u4, sel, (((1,), (0,)), ((), ())),
                          preferred_element_type=jnp.float32)
    r_ref[0, 0] = tcv[:, 0:1]


def _phase1(cand, q3, m3, t3):
    B, N, D = cand.shape
    nb = N // BN
    rg = BN // 128
    return pl.pallas_call(
        _phase1_body,
        grid=(B, nb),
        in_specs=[
            pl.BlockSpec((1, BN, D), lambda b, j: (b, j, 0)),
            pl.BlockSpec((1, 1, D), lambda b, j: (b, 0, 0)),
            pl.BlockSpec((1, 1, D), lambda b, j: (b, 0, 0)),
            pl.BlockSpec((1, 1, D), lambda b, j: (b, 0, 0)),
        ],
        out_specs=[
            pl.BlockSpec((1, 1, rg, 128), lambda b, j: (b, j, 0, 0)),
            pl.BlockSpec((1, 1, BN, 1), lambda b, j: (b, j, 0, 0)),
        ],
        out_shape=[
            jax.ShapeDtypeStruct((B, nb, rg, 128), jnp.float32),
            jax.ShapeDtypeStruct((B, nb, BN, 1), jnp.float32),
        ],
    )(cand, q3, m3, t3)


def _phase2_body(n_total, d_ref, r_ref, m_ref, t_ref, w_ref, b_ref,
                 scores_ref, idx_ref, gidx_ref, x_ref):
    nb, bn = x_ref.shape
    dist2d = d_ref[0].reshape(nb, bn)
    corr2d = dist2d + 0.1 * r_ref[0].reshape(nb, bn)
    x_ref[...] = dist2d

    row_i = lax.broadcasted_iota(jnp.int32, (nb, bn), 0)
    col_i = lax.broadcasted_iota(jnp.int32, (nb, bn), 1)
    gidx2d = row_i * bn + col_i
    lane32 = lax.broadcasted_iota(jnp.int32, (1, 32), 1)
    INF = jnp.float32(jnp.inf)
    BIGI = jnp.int32(2**30)

    def body(k, carry):
        vacc, iacc, cacc = carry
        x = x_ref[...]
        mv = jnp.min(x)
        ci = jnp.min(jnp.where(x == mv, gidx2d, BIGI))
        sel = gidx2d == ci
        cv = jnp.sum(jnp.where(sel, corr2d, 0.0))
        onehot = lane32 == k
        vacc = jnp.where(onehot, mv, vacc)
        iacc = jnp.where(onehot, ci, iacc)
        cacc = jnp.where(onehot, cv, cacc)
        x_ref[...] = jnp.where(sel, INF, x)
        return vacc, iacc, cacc

    vacc = jnp.zeros((1, 32), jnp.float32)
    iacc = jnp.zeros((1, 32), jnp.int32)
    cacc = jnp.zeros((1, 32), jnp.float32)
    vacc, iacc, cacc = lax.fori_loop(0, 32, body, (vacc, iacc, cacc))

    m_row = m_ref[0]
    t_row = t_ref[0]
    D = m_row.shape[1]
    w = w_ref[...]
    bs = (jnp.sum(w[:, :D] * m_row, axis=1)
          + jnp.sum(w[:, D:] * t_row, axis=1) + b_ref[0])
    iota6 = lax.iota(jnp.int32, 6)

    def bs_at(j):
        return jnp.sum(jnp.where(iota6 == j, bs, 0.0))

    d16 = dist2d[0:1, 0:16]
    c16 = corr2d[0:1, 0:16]

    eps = jnp.float32(0.001)
    ch0_near = 1.0 / (vacc + eps)
    ch1_near = 1.0 / (jnp.abs(cacc) + eps)
    ch0_16 = 1.0 / (d16 + eps)
    ch1_16 = 1.0 / (jnp.abs(c16) + eps)

    ch0 = jnp.concatenate([ch0_near, ch0_16, ch0_16], axis=1)
    ch1 = jnp.concatenate([ch1_near, ch1_16, ch1_16], axis=1)
    ones64 = jnp.ones((1, 64), jnp.float32)
    ch2 = ones64 * bs_at(2)
    ch3 = ones64 * bs_at(3)
    ch4 = ones64 * bs_at(4)
    ch5 = lax.broadcasted_iota(jnp.int32, (1, 64), 1).astype(jnp.float32) * (
        1.0 / 64.0)

    scores = jnp.concatenate(
        [ch[0, :, None] for ch in (ch0, ch1, ch2, ch3, ch4, ch5)], axis=1)
    scores_ref[0] = scores

    iota16 = lax.broadcasted_iota(jnp.int32, (1, 16), 1)
    idx64 = jnp.concatenate([iacc, iota16, iota16], axis=1)
    idx_ref[0, 0] = idx64[0]
    gidx_ref[0, 0] = idx64[0] + pl.program_id(0) * n_total


def _phase2(dist4, corr4, m3, t3, W, b2):
    B, nb0, rg, bn = dist4.shape
    nb = nb0 * rg
    D = m3.shape[2]
    N = nb * bn
    return pl.pallas_call(
        functools.partial(_phase2_body, N),
        grid=(B,),
        in_specs=[
            pl.BlockSpec((1, nb0, rg, bn), lambda b: (b, 0, 0, 0)),
            pl.BlockSpec((1, nb0, rg, bn), lambda b: (b, 0, 0, 0)),
            pl.BlockSpec((1, 1, D), lambda b: (b, 0, 0)),
            pl.BlockSpec((1, 1, D), lambda b: (b, 0, 0)),
            pl.BlockSpec((6, 2 * D), lambda b: (0, 0)),
            pl.BlockSpec((1, 6), lambda b: (0, 0)),
        ],
        out_specs=[
            pl.BlockSpec((1, 64, 6), lambda b: (b, 0, 0)),
            pl.BlockSpec((1, 1, 64), lambda b: (b, 0, 0)),
            pl.BlockSpec((1, 1, 64), lambda b: (b, 0, 0)),
        ],
        out_shape=[
            jax.ShapeDtypeStruct((B, 64, 6), jnp.float32),
            jax.ShapeDtypeStruct((B, 1, 64), jnp.int32),
            jax.ShapeDtypeStruct((B, 1, 64), jnp.int32),
        ],
        scratch_shapes=[pltpu.VMEM((nb, bn), jnp.float32)],
    )(dist4, corr4, m3, t3, W, b2)


def _sc_gather(table, gidx):
    G = gidx.shape[0]
    D = table.shape[1]
    info = plsc.get_sparse_core_info()
    nc, ns = info.num_cores, info.num_subcores
    nw = nc * ns
    bpw = G // nw
    mesh = plsc.VectorSubcoreMesh(core_axis_name="c", subcore_axis_name="s")

    @functools.partial(
        pl.kernel, mesh=mesh,
        out_type=jax.ShapeDtypeStruct((G, D), jnp.float32),
        scratch_types=[
            pltpu.VMEM((bpw,), jnp.int32),
            pltpu.VMEM((bpw, D), jnp.float32),
            pltpu.SemaphoreType.DMA,
        ],
    )
    def k(table_hbm, idx_hbm, out_hbm, idx_v, rows_v, sem):
        wid = lax.axis_index("s") * nc + lax.axis_index("c")
        base = wid * bpw
        pltpu.sync_copy(idx_hbm.at[pl.ds(base, bpw)], idx_v)
        pltpu.async_copy(table_hbm.at[idx_v], rows_v, sem).wait()
        pltpu.sync_copy(rows_v, out_hbm.at[pl.ds(base, bpw)])

    return k(table, gidx)


def kernel(query_embedding, candidate_embeddings, metric, transport, W, b):
    B, N, D = candidate_embeddings.shape
    q3 = query_embedding.reshape(B, 1, D)
    m3 = metric.reshape(B, 1, D)
    t3 = transport.reshape(B, 1, D)
    b2 = b.reshape(1, 6)

    dist4, tc4 = _phase1(candidate_embeddings, q3, m3, t3)
    tc4 = tc4.reshape(dist4.shape)
    scores, idx3, gidx3 = _phase2(dist4, tc4, m3, t3, W, b2)

    flat = candidate_embeddings.reshape(B * N, D)
    rows = _sc_gather(flat, gidx3.reshape(B * 64))
    selected_embeddings = rows.reshape(B, 64, D)
    selected_indices = idx3.reshape(B, 64)
    return selected_embeddings, scores, selected_indices
